# baseline (speedup 1.0000x reference)
import math

import jax
import jax.numpy as jnp
import numpy as np
from jax.experimental import pallas as pl
from jax.experimental.pallas import tpu as pltpu

_B, _C, _H, _W = 8, 384, 32, 32
_HEADS, _POINTS = 8, 4
_HD = _C // _HEADS
_NQ = _H * _W


def _make_pe_flat():
    d_model, h, w = _C, _H, _W
    pe = np.zeros((d_model, h, w), dtype=np.float64)
    y_pos = np.cumsum(np.ones((h, w)), axis=0)
    x_pos = np.cumsum(np.ones((h, w)), axis=1)
    div = np.exp(np.arange(0, d_model // 2, 2) * (-math.log(10000.0) / (d_model // 2)))
    div = div[:, None, None]
    pe[0::4] = np.sin(x_pos[None] * div)
    pe[1::4] = np.cos(x_pos[None] * div)
    pe[2::4] = np.sin(y_pos[None] * div)
    pe[3::4] = np.cos(y_pos[None] * div)
    return pe.reshape(d_model, h * w).T.astype(np.float32)


def _make_ref_points():
    row_ = np.linspace(0.0, 1.0, _W)
    col_ = np.linspace(0.0, 1.0, _H)
    r, cg = np.meshgrid(row_, col_, indexing="ij")
    ref = np.stack((cg, r), -1).reshape(-1, 2)
    return ref.astype(np.float32)


_PE = _make_pe_flat()
_REF = _make_ref_points()
_E = np.repeat(np.eye(_H, dtype=np.float32), _W, axis=1)
_T = np.tile(np.eye(_W, dtype=np.float32), (1, _H))


def _body(qf_ref, vf_ref, pe_ref, ref_ref, WvT_ref, bv_ref, WoT_ref, bo_ref,
          WaT_ref, ba_ref, WpT_ref, bp_ref, out_ref, A_ref):
    qf = qf_ref[0]
    vf = vf_ref[0]
    qk = qf + pe_ref[...]

    v = jnp.dot(vf, WvT_ref[...], preferred_element_type=jnp.float32) + bv_ref[...]
    v16 = v.astype(jnp.bfloat16)
    off = jnp.dot(qk, WoT_ref[...], preferred_element_type=jnp.float32) + bo_ref[...]
    awl = jnp.dot(qk, WaT_ref[...], preferred_element_type=jnp.float32) + ba_ref[...]

    refx = ref_ref[:, 0:1]
    refy = ref_ref[:, 1:2]
    iota = jax.lax.broadcasted_iota(jnp.int32, (_NQ, _W), 1)

    acc = None
    for hh in range(_HEADS):
        a = awl[:, 4 * hh:4 * hh + 4]
        m = jnp.max(a, axis=1, keepdims=True)
        e = jnp.exp(a - m)
        aw = e / jnp.sum(e, axis=1, keepdims=True)

        v_h = v16[:, _HD * hh:_HD * (hh + 1)]

        for p in range(_POINTS):
            ox = off[:, 8 * hh + 2 * p:8 * hh + 2 * p + 1]
            oy = off[:, 8 * hh + 2 * p + 1:8 * hh + 2 * p + 2]
            xpx = (refx + ox * (1.0 / _W)) * float(_W) - 0.5
            ypx = (refy + oy * (1.0 / _H)) * float(_H) - 0.5
            x0 = jnp.floor(xpx)
            y0 = jnp.floor(ypx)
            wx1 = xpx - x0
            wx0 = 1.0 - wx1
            wy1 = ypx - y0
            wy0 = 1.0 - wy1
            vx0 = ((x0 >= 0.0) & (x0 <= _W - 1.0)).astype(jnp.float32)
            vx1 = ((x0 >= -1.0) & (x0 <= _W - 2.0)).astype(jnp.float32)
            vy0 = ((y0 >= 0.0) & (y0 <= _H - 1.0)).astype(jnp.float32)
            vy1 = ((y0 >= -1.0) & (y0 <= _H - 2.0)).astype(jnp.float32)
            x0c = jnp.clip(x0, 0.0, _W - 1.0).astype(jnp.int32)
            x1c = jnp.clip(x0 + 1.0, 0.0, _W - 1.0).astype(jnp.int32)
            y0c = jnp.clip(y0, 0.0, _H - 1.0).astype(jnp.int32)
            y1c = jnp.clip(y0 + 1.0, 0.0, _H - 1.0).astype(jnp.int32)

            Cm = (jnp.where(iota == x0c, wx0 * vx0, 0.0)
                  + jnp.where(iota == x1c, wx1 * vx1, 0.0))
            Rm = (jnp.where(iota == y0c, wy0 * vy0, 0.0)
                  + jnp.where(iota == y1c, wy1 * vy1, 0.0)) * aw[:, p:p + 1]

            term = (jnp.repeat(Rm.astype(jnp.bfloat16), _W, axis=1)
                    * jnp.tile(Cm.astype(jnp.bfloat16), (1, _H)))
            if p == 0:
                A_ref[...] = term
            else:
                A_ref[...] += term

        out_h = jnp.dot(A_ref[...], v_h, preferred_element_type=jnp.float32)
        part = jnp.dot(out_h, WpT_ref[_HD * hh:_HD * (hh + 1), :],
                       preferred_element_type=jnp.float32)
        acc = part if acc is None else acc + part

    out_ref[0] = acc + bp_ref[...] + 2.0 * qk


@jax.jit
def kernel(query, value, Wv, bv, Wo, bo, Wa, ba, Wp, bp):
    b, c, h, w = query.shape
    nq = h * w
    qf = query.reshape(b, c, nq).transpose(0, 2, 1)
    vf = value.reshape(b, c, nq).transpose(0, 2, 1)

    out = pl.pallas_call(
        _body,
        grid=(b,),
        in_specs=[
            pl.BlockSpec((1, _NQ, _C), lambda i: (i, 0, 0)),
            pl.BlockSpec((1, _NQ, _C), lambda i: (i, 0, 0)),
            pl.BlockSpec((_NQ, _C), lambda i: (0, 0)),
            pl.BlockSpec((_NQ, 2), lambda i: (0, 0)),
            pl.BlockSpec((_C, _C), lambda i: (0, 0)),
            pl.BlockSpec((1, _C), lambda i: (0, 0)),
            pl.BlockSpec((_C, _HEADS * _POINTS * 2), lambda i: (0, 0)),
            pl.BlockSpec((1, _HEADS * _POINTS * 2), lambda i: (0, 0)),
            pl.BlockSpec((_C, _HEADS * _POINTS), lambda i: (0, 0)),
            pl.BlockSpec((1, _HEADS * _POINTS), lambda i: (0, 0)),
            pl.BlockSpec((_C, _C), lambda i: (0, 0)),
            pl.BlockSpec((1, _C), lambda i: (0, 0)),
        ],
        out_specs=pl.BlockSpec((1, _NQ, _C), lambda i: (i, 0, 0)),
        out_shape=jax.ShapeDtypeStruct((b, _NQ, _C), jnp.float32),
        scratch_shapes=[pltpu.VMEM((_NQ, _NQ), jnp.bfloat16)],
        compiler_params=pltpu.CompilerParams(
            dimension_semantics=("parallel",)),
    )(qf, vf, jnp.asarray(_PE), jnp.asarray(_REF),
      Wv.T, bv.reshape(1, -1), Wo.T, bo.reshape(1, -1),
      Wa.T, ba.reshape(1, -1), Wp.T, bp.reshape(1, -1))

    return out.transpose(0, 2, 1).reshape(b, c, h, w)


# speedup vs baseline: 2.8400x; 2.8400x over previous
import math

import jax
import jax.numpy as jnp
import numpy as np
from jax.experimental import pallas as pl
from jax.experimental.pallas import tpu as pltpu

_B, _C, _H, _W = 8, 384, 32, 32
_HEADS, _POINTS = 8, 4
_HD = _C // _HEADS
_NQ = _H * _W


def _make_pe_flat():
    d_model, h, w = _C, _H, _W
    pe = np.zeros((d_model, h, w), dtype=np.float64)
    y_pos = np.cumsum(np.ones((h, w)), axis=0)
    x_pos = np.cumsum(np.ones((h, w)), axis=1)
    div = np.exp(np.arange(0, d_model // 2, 2) * (-math.log(10000.0) / (d_model // 2)))
    div = div[:, None, None]
    pe[0::4] = np.sin(x_pos[None] * div)
    pe[1::4] = np.cos(x_pos[None] * div)
    pe[2::4] = np.sin(y_pos[None] * div)
    pe[3::4] = np.cos(y_pos[None] * div)
    return pe.reshape(d_model, h * w).T.astype(np.float32)


def _make_ref_points():
    row_ = np.linspace(0.0, 1.0, _W)
    col_ = np.linspace(0.0, 1.0, _H)
    r, cg = np.meshgrid(row_, col_, indexing="ij")
    ref = np.stack((cg, r), -1).reshape(-1, 2)
    return ref.astype(np.float32)


_PE = _make_pe_flat()
_REF = _make_ref_points()
_E = np.repeat(np.eye(_H, dtype=np.float32), _W, axis=1)
_T = np.tile(np.eye(_W, dtype=np.float32), (1, _H))


def _body(qf_ref, vf_ref, pe_ref, ref_ref, WvT_ref, bv_ref, WoT_ref, bo_ref,
          WaT_ref, ba_ref, WpT_ref, bp_ref, E_ref, T_ref, out_ref, A_ref):
    qf = qf_ref[0]
    vf = vf_ref[0]
    qk = qf + pe_ref[...]

    v = jnp.dot(vf, WvT_ref[...], preferred_element_type=jnp.float32) + bv_ref[...]
    v16 = v.astype(jnp.bfloat16)
    off = jnp.dot(qk, WoT_ref[...], preferred_element_type=jnp.float32) + bo_ref[...]
    awl = jnp.dot(qk, WaT_ref[...], preferred_element_type=jnp.float32) + ba_ref[...]

    refx = ref_ref[:, 0:1]
    refy = ref_ref[:, 1:2]
    iota = jax.lax.broadcasted_iota(jnp.int32, (_NQ, _W), 1)

    acc = None
    for hh in range(_HEADS):
        a = awl[:, 4 * hh:4 * hh + 4]
        m = jnp.max(a, axis=1, keepdims=True)
        e = jnp.exp(a - m)
        aw = e / jnp.sum(e, axis=1, keepdims=True)

        v_h = v16[:, _HD * hh:_HD * (hh + 1)]

        for p in range(_POINTS):
            ox = off[:, 8 * hh + 2 * p:8 * hh + 2 * p + 1]
            oy = off[:, 8 * hh + 2 * p + 1:8 * hh + 2 * p + 2]
            xpx = (refx + ox * (1.0 / _W)) * float(_W) - 0.5
            ypx = (refy + oy * (1.0 / _H)) * float(_H) - 0.5
            x0 = jnp.floor(xpx)
            y0 = jnp.floor(ypx)
            wx1 = xpx - x0
            wx0 = 1.0 - wx1
            wy1 = ypx - y0
            wy0 = 1.0 - wy1
            vx0 = ((x0 >= 0.0) & (x0 <= _W - 1.0)).astype(jnp.float32)
            vx1 = ((x0 >= -1.0) & (x0 <= _W - 2.0)).astype(jnp.float32)
            vy0 = ((y0 >= 0.0) & (y0 <= _H - 1.0)).astype(jnp.float32)
            vy1 = ((y0 >= -1.0) & (y0 <= _H - 2.0)).astype(jnp.float32)
            x0c = jnp.clip(x0, 0.0, _W - 1.0).astype(jnp.int32)
            x1c = jnp.clip(x0 + 1.0, 0.0, _W - 1.0).astype(jnp.int32)
            y0c = jnp.clip(y0, 0.0, _H - 1.0).astype(jnp.int32)
            y1c = jnp.clip(y0 + 1.0, 0.0, _H - 1.0).astype(jnp.int32)

            Cm = (jnp.where(iota == x0c, wx0 * vx0, 0.0)
                  + jnp.where(iota == x1c, wx1 * vx1, 0.0))
            Rm = (jnp.where(iota == y0c, wy0 * vy0, 0.0)
                  + jnp.where(iota == y1c, wy1 * vy1, 0.0)) * aw[:, p:p + 1]

            term = (jnp.dot(Rm.astype(jnp.bfloat16), E_ref[...],
                            preferred_element_type=jnp.float32)
                    * jnp.dot(Cm.astype(jnp.bfloat16), T_ref[...],
                              preferred_element_type=jnp.float32))
            if p == 0:
                A_ref[...] = term.astype(jnp.bfloat16)
            else:
                A_ref[...] += term.astype(jnp.bfloat16)

        out_h = jnp.dot(A_ref[...], v_h, preferred_element_type=jnp.float32)
        part = jnp.dot(out_h, WpT_ref[_HD * hh:_HD * (hh + 1), :],
                       preferred_element_type=jnp.float32)
        acc = part if acc is None else acc + part

    out_ref[0] = acc + bp_ref[...] + 2.0 * qk


@jax.jit
def kernel(query, value, Wv, bv, Wo, bo, Wa, ba, Wp, bp):
    b, c, h, w = query.shape
    nq = h * w
    qf = query.reshape(b, c, nq).transpose(0, 2, 1)
    vf = value.reshape(b, c, nq).transpose(0, 2, 1)

    out = pl.pallas_call(
        _body,
        grid=(b,),
        in_specs=[
            pl.BlockSpec((1, _NQ, _C), lambda i: (i, 0, 0)),
            pl.BlockSpec((1, _NQ, _C), lambda i: (i, 0, 0)),
            pl.BlockSpec((_NQ, _C), lambda i: (0, 0)),
            pl.BlockSpec((_NQ, 2), lambda i: (0, 0)),
            pl.BlockSpec((_C, _C), lambda i: (0, 0)),
            pl.BlockSpec((1, _C), lambda i: (0, 0)),
            pl.BlockSpec((_C, _HEADS * _POINTS * 2), lambda i: (0, 0)),
            pl.BlockSpec((1, _HEADS * _POINTS * 2), lambda i: (0, 0)),
            pl.BlockSpec((_C, _HEADS * _POINTS), lambda i: (0, 0)),
            pl.BlockSpec((1, _HEADS * _POINTS), lambda i: (0, 0)),
            pl.BlockSpec((_C, _C), lambda i: (0, 0)),
            pl.BlockSpec((1, _C), lambda i: (0, 0)),
            pl.BlockSpec((_H, _NQ), lambda i: (0, 0)),
            pl.BlockSpec((_W, _NQ), lambda i: (0, 0)),
        ],
        out_specs=pl.BlockSpec((1, _NQ, _C), lambda i: (i, 0, 0)),
        out_shape=jax.ShapeDtypeStruct((b, _NQ, _C), jnp.float32),
        scratch_shapes=[pltpu.VMEM((_NQ, _NQ), jnp.bfloat16)],
        compiler_params=pltpu.CompilerParams(
            dimension_semantics=("parallel",)),
    )(qf, vf, jnp.asarray(_PE), jnp.asarray(_REF),
      Wv.T, bv.reshape(1, -1), Wo.T, bo.reshape(1, -1),
      Wa.T, ba.reshape(1, -1), Wp.T, bp.reshape(1, -1),
      jnp.asarray(_E, dtype=jnp.bfloat16), jnp.asarray(_T, dtype=jnp.bfloat16))

    return out.transpose(0, 2, 1).reshape(b, c, h, w)


# speedup vs baseline: 3.4817x; 1.2259x over previous
import math

import jax
import jax.numpy as jnp
import numpy as np
from jax.experimental import pallas as pl
from jax.experimental.pallas import tpu as pltpu

_B, _C, _H, _W = 8, 384, 32, 32
_HEADS, _POINTS = 8, 4
_HD = _C // _HEADS
_NQ = _H * _W


def _make_pe_flat():
    d_model, h, w = _C, _H, _W
    pe = np.zeros((d_model, h, w), dtype=np.float64)
    y_pos = np.cumsum(np.ones((h, w)), axis=0)
    x_pos = np.cumsum(np.ones((h, w)), axis=1)
    div = np.exp(np.arange(0, d_model // 2, 2) * (-math.log(10000.0) / (d_model // 2)))
    div = div[:, None, None]
    pe[0::4] = np.sin(x_pos[None] * div)
    pe[1::4] = np.cos(x_pos[None] * div)
    pe[2::4] = np.sin(y_pos[None] * div)
    pe[3::4] = np.cos(y_pos[None] * div)
    return pe.reshape(d_model, h * w).T.astype(np.float32)


def _make_ref_points():
    row_ = np.linspace(0.0, 1.0, _W)
    col_ = np.linspace(0.0, 1.0, _H)
    r, cg = np.meshgrid(row_, col_, indexing="ij")
    ref = np.stack((cg, r), -1).reshape(-1, 2)
    return ref.astype(np.float32)


_PE = _make_pe_flat()
_REF = _make_ref_points()
_E32 = np.repeat(np.eye(_H, dtype=np.float32), _W, axis=1)
_T32 = np.tile(np.eye(_W, dtype=np.float32), (1, _H))
_EPS = []
_TPS = []
for _p in range(_POINTS):
    _e = np.zeros((4 * _H, _NQ), dtype=np.float32)
    _t = np.zeros((4 * _W, _NQ), dtype=np.float32)
    _e[32 * _p:32 * _p + 32] = _E32
    _t[32 * _p:32 * _p + 32] = _T32
    _EPS.append(_e)
    _TPS.append(_t)
_E4B = np.zeros((_POINTS, 4 * _W), dtype=np.float32)
for _p in range(_POINTS):
    _E4B[_p, 32 * _p:32 * _p + 32] = 1.0
_S4 = np.kron(np.eye(_HEADS, dtype=np.float32), np.ones((4, 4), np.float32))
_PERMO = ([8 * h + 2 * p for h in range(_HEADS) for p in range(_POINTS)]
          + [8 * h + 2 * p + 1 for h in range(_HEADS) for p in range(_POINTS)])


def _body(qf_ref, vf_ref, pe_ref, ref_ref, WvT_ref, bv_ref, WoT_ref, bo_ref,
          WaT_ref, ba_ref, WpT_ref, bp_ref, E4B_ref, S4_ref,
          E0_ref, E1_ref, E2_ref, E3_ref, T0_ref, T1_ref, T2_ref, T3_ref,
          out_ref, A_ref):
    qf = qf_ref[0]
    vf = vf_ref[0]
    qk = qf + pe_ref[...]

    v = jnp.dot(vf, WvT_ref[...], preferred_element_type=jnp.float32) + bv_ref[...]
    v16 = v.astype(jnp.bfloat16)
    off = jnp.dot(qk, WoT_ref[...], preferred_element_type=jnp.float32) + bo_ref[...]
    awl = jnp.dot(qk, WaT_ref[...], preferred_element_type=jnp.float32) + ba_ref[...]

    rowmax = jnp.max(awl, axis=1, keepdims=True)
    ex = jnp.exp(awl - rowmax)
    aw_all = ex / jnp.dot(ex, S4_ref[...], preferred_element_type=jnp.float32)

    refx = ref_ref[:, 0:1]
    refy = ref_ref[:, 1:2]
    ik = jax.lax.broadcasted_iota(jnp.int32, (_NQ, 4 * _W), 1).astype(jnp.float32)
    ik = ik - float(_W) * jnp.floor(ik * (1.0 / _W))
    E4B = E4B_ref[...]
    Eps = [E0_ref[...], E1_ref[...], E2_ref[...], E3_ref[...]]
    Tps = [T0_ref[...], T1_ref[...], T2_ref[...], T3_ref[...]]

    acc = None
    for hh in range(_HEADS):
        oxc = jnp.dot(off[:, 4 * hh:4 * hh + 4], E4B,
                      preferred_element_type=jnp.float32)
        oyc = jnp.dot(off[:, 32 + 4 * hh:32 + 4 * hh + 4], E4B,
                      preferred_element_type=jnp.float32)
        awc = jnp.dot(aw_all[:, 4 * hh:4 * hh + 4], E4B,
                      preferred_element_type=jnp.float32)

        xpx = (refx + oxc * (1.0 / _W)) * float(_W) - 0.5
        ypx = (refy + oyc * (1.0 / _H)) * float(_H) - 0.5
        x0 = jnp.floor(xpx)
        y0 = jnp.floor(ypx)
        wx1 = xpx - x0
        wx0 = 1.0 - wx1
        wy1 = ypx - y0
        wy0 = 1.0 - wy1
        vx0 = ((x0 >= 0.0) & (x0 <= _W - 1.0)).astype(jnp.float32)
        vx1 = ((x0 >= -1.0) & (x0 <= _W - 2.0)).astype(jnp.float32)
        vy0 = ((y0 >= 0.0) & (y0 <= _H - 1.0)).astype(jnp.float32)
        vy1 = ((y0 >= -1.0) & (y0 <= _H - 2.0)).astype(jnp.float32)
        x0c = jnp.clip(x0, 0.0, _W - 1.0)
        x1c = jnp.clip(x0 + 1.0, 0.0, _W - 1.0)
        y0c = jnp.clip(y0, 0.0, _H - 1.0)
        y1c = jnp.clip(y0 + 1.0, 0.0, _H - 1.0)

        Cm = (jnp.where(ik == x0c, wx0 * vx0, 0.0)
              + jnp.where(ik == x1c, wx1 * vx1, 0.0))
        Rm = (jnp.where(ik == y0c, wy0 * vy0, 0.0)
              + jnp.where(ik == y1c, wy1 * vy1, 0.0)) * awc
        Rm16 = Rm.astype(jnp.bfloat16)
        Cm16 = Cm.astype(jnp.bfloat16)

        for p in range(_POINTS):
            term = (jnp.dot(Rm16, Eps[p], preferred_element_type=jnp.float32)
                    * jnp.dot(Cm16, Tps[p], preferred_element_type=jnp.float32))
            if p == 0:
                A_ref[...] = term.astype(jnp.bfloat16)
            else:
                A_ref[...] += term.astype(jnp.bfloat16)

        v_h = v16[:, _HD * hh:_HD * (hh + 1)]
        out_h = jnp.dot(A_ref[...], v_h, preferred_element_type=jnp.float32)
        part = jnp.dot(out_h, WpT_ref[_HD * hh:_HD * (hh + 1), :],
                       preferred_element_type=jnp.float32)
        acc = part if acc is None else acc + part

    out_ref[0] = acc + bp_ref[...] + 2.0 * qk


@jax.jit
def kernel(query, value, Wv, bv, Wo, bo, Wa, ba, Wp, bp):
    b, c, h, w = query.shape
    nq = h * w
    qf = query.reshape(b, c, nq).transpose(0, 2, 1)
    vf = value.reshape(b, c, nq).transpose(0, 2, 1)
    permo = jnp.asarray(_PERMO)
    WoT = Wo.T[:, permo]
    boP = bo[permo].reshape(1, -1)

    bcast = lambda i: (0, 0)
    out = pl.pallas_call(
        _body,
        grid=(b,),
        in_specs=[
            pl.BlockSpec((1, _NQ, _C), lambda i: (i, 0, 0)),
            pl.BlockSpec((1, _NQ, _C), lambda i: (i, 0, 0)),
            pl.BlockSpec((_NQ, _C), bcast),
            pl.BlockSpec((_NQ, 2), bcast),
            pl.BlockSpec((_C, _C), bcast),
            pl.BlockSpec((1, _C), bcast),
            pl.BlockSpec((_C, _HEADS * _POINTS * 2), bcast),
            pl.BlockSpec((1, _HEADS * _POINTS * 2), bcast),
            pl.BlockSpec((_C, _HEADS * _POINTS), bcast),
            pl.BlockSpec((1, _HEADS * _POINTS), bcast),
            pl.BlockSpec((_C, _C), bcast),
            pl.BlockSpec((1, _C), bcast),
            pl.BlockSpec((_POINTS, 4 * _W), bcast),
            pl.BlockSpec((_HEADS * _POINTS, _HEADS * _POINTS), bcast),
            pl.BlockSpec((4 * _H, _NQ), bcast),
            pl.BlockSpec((4 * _H, _NQ), bcast),
            pl.BlockSpec((4 * _H, _NQ), bcast),
            pl.BlockSpec((4 * _H, _NQ), bcast),
            pl.BlockSpec((4 * _W, _NQ), bcast),
            pl.BlockSpec((4 * _W, _NQ), bcast),
            pl.BlockSpec((4 * _W, _NQ), bcast),
            pl.BlockSpec((4 * _W, _NQ), bcast),
        ],
        out_specs=pl.BlockSpec((1, _NQ, _C), lambda i: (i, 0, 0)),
        out_shape=jax.ShapeDtypeStruct((b, _NQ, _C), jnp.float32),
        scratch_shapes=[pltpu.VMEM((_NQ, _NQ), jnp.bfloat16)],
        compiler_params=pltpu.CompilerParams(
            dimension_semantics=("parallel",)),
    )(qf, vf, jnp.asarray(_PE), jnp.asarray(_REF),
      Wv.T, bv.reshape(1, -1), WoT, boP,
      Wa.T, ba.reshape(1, -1), Wp.T, bp.reshape(1, -1),
      jnp.asarray(_E4B), jnp.asarray(_S4),
      *[jnp.asarray(e, dtype=jnp.bfloat16) for e in _EPS],
      *[jnp.asarray(t, dtype=jnp.bfloat16) for t in _TPS])

    return out.transpose(0, 2, 1).reshape(b, c, h, w)


# speedup vs baseline: 3.4960x; 1.0041x over previous
import math

import jax
import jax.numpy as jnp
import numpy as np
from jax.experimental import pallas as pl
from jax.experimental.pallas import tpu as pltpu

_B, _C, _H, _W = 8, 384, 32, 32
_HEADS, _POINTS = 8, 4
_HD = _C // _HEADS
_NQ = _H * _W


def _make_pe_flat():
    d_model, h, w = _C, _H, _W
    pe = np.zeros((d_model, h, w), dtype=np.float64)
    y_pos = np.cumsum(np.ones((h, w)), axis=0)
    x_pos = np.cumsum(np.ones((h, w)), axis=1)
    div = np.exp(np.arange(0, d_model // 2, 2) * (-math.log(10000.0) / (d_model // 2)))
    div = div[:, None, None]
    pe[0::4] = np.sin(x_pos[None] * div)
    pe[1::4] = np.cos(x_pos[None] * div)
    pe[2::4] = np.sin(y_pos[None] * div)
    pe[3::4] = np.cos(y_pos[None] * div)
    return pe.reshape(d_model, h * w).T.astype(np.float32)


def _make_ref_points():
    row_ = np.linspace(0.0, 1.0, _W)
    col_ = np.linspace(0.0, 1.0, _H)
    r, cg = np.meshgrid(row_, col_, indexing="ij")
    ref = np.stack((cg, r), -1).reshape(-1, 2)
    return ref.astype(np.float32)


_PE = _make_pe_flat()
_REF = _make_ref_points()
_E32 = np.repeat(np.eye(_H, dtype=np.float32), _W, axis=1)
_T32 = np.tile(np.eye(_W, dtype=np.float32), (1, _H))
_EPS = []
_TPS = []
for _p in range(_POINTS):
    _e = np.zeros((4 * _H, _NQ), dtype=np.float32)
    _t = np.zeros((4 * _W, _NQ), dtype=np.float32)
    _e[32 * _p:32 * _p + 32] = _E32
    _t[32 * _p:32 * _p + 32] = _T32
    _EPS.append(_e)
    _TPS.append(_t)
_E4B = np.zeros((_POINTS, 4 * _W), dtype=np.float32)
for _p in range(_POINTS):
    _E4B[_p, 32 * _p:32 * _p + 32] = 1.0
_S4 = np.kron(np.eye(_HEADS, dtype=np.float32), np.ones((4, 4), np.float32))
_PERMO = ([8 * h + 2 * p for h in range(_HEADS) for p in range(_POINTS)]
          + [8 * h + 2 * p + 1 for h in range(_HEADS) for p in range(_POINTS)])


def _body(qf_ref, vf_ref, pe_ref, ref_ref, WvT_ref, bv_ref, WoT_ref, bo_ref,
          WaT_ref, ba_ref, WpT_ref, bp_ref, E4B_ref, S4_ref,
          E0_ref, E1_ref, E2_ref, E3_ref, T0_ref, T1_ref, T2_ref, T3_ref,
          out_ref, A_ref):
    qf = qf_ref[0]
    vf = vf_ref[0]
    qk = qf + pe_ref[...]

    v = jnp.dot(vf, WvT_ref[...], preferred_element_type=jnp.float32) + bv_ref[...]
    v16 = v.astype(jnp.bfloat16)
    off = jnp.dot(qk, WoT_ref[...], preferred_element_type=jnp.float32) + bo_ref[...]
    awl = jnp.dot(qk, WaT_ref[...], preferred_element_type=jnp.float32) + ba_ref[...]

    rowmax = jnp.max(awl, axis=1, keepdims=True)
    ex = jnp.exp(awl - rowmax)
    aw_all = ex / jnp.dot(ex, S4_ref[...], preferred_element_type=jnp.float32)

    refx = ref_ref[:, 0:1]
    refy = ref_ref[:, 1:2]
    ik = jax.lax.broadcasted_iota(jnp.int32, (_NQ, 4 * _W), 1).astype(jnp.float32)
    ik = ik - float(_W) * jnp.floor(ik * (1.0 / _W))
    E4B = E4B_ref[...]
    Eps = [E0_ref[...], E1_ref[...], E2_ref[...], E3_ref[...]]
    Tps = [T0_ref[...], T1_ref[...], T2_ref[...], T3_ref[...]]

    acc = None
    for hh in range(_HEADS):
        oxc = jnp.dot(off[:, 4 * hh:4 * hh + 4], E4B,
                      preferred_element_type=jnp.float32)
        oyc = jnp.dot(off[:, 32 + 4 * hh:32 + 4 * hh + 4], E4B,
                      preferred_element_type=jnp.float32)
        awc = jnp.dot(aw_all[:, 4 * hh:4 * hh + 4], E4B,
                      preferred_element_type=jnp.float32)

        xpx = (refx + oxc * (1.0 / _W)) * float(_W) - 0.5
        ypx = (refy + oyc * (1.0 / _H)) * float(_H) - 0.5
        x0 = jnp.floor(xpx)
        y0 = jnp.floor(ypx)
        wx1 = xpx - x0
        wx0 = 1.0 - wx1
        wy1 = ypx - y0
        wy0 = 1.0 - wy1
        vx0 = ((x0 >= 0.0) & (x0 <= _W - 1.0)).astype(jnp.float32)
        vx1 = ((x0 >= -1.0) & (x0 <= _W - 2.0)).astype(jnp.float32)
        vy0 = ((y0 >= 0.0) & (y0 <= _H - 1.0)).astype(jnp.float32)
        vy1 = ((y0 >= -1.0) & (y0 <= _H - 2.0)).astype(jnp.float32)
        x0c = jnp.clip(x0, 0.0, _W - 1.0)
        x1c = jnp.clip(x0 + 1.0, 0.0, _W - 1.0)
        y0c = jnp.clip(y0, 0.0, _H - 1.0)
        y1c = jnp.clip(y0 + 1.0, 0.0, _H - 1.0)

        Cm = (jnp.where(ik == x0c, wx0 * vx0, 0.0)
              + jnp.where(ik == x1c, wx1 * vx1, 0.0))
        Rm = (jnp.where(ik == y0c, wy0 * vy0, 0.0)
              + jnp.where(ik == y1c, wy1 * vy1, 0.0)) * awc
        Rm16 = Rm.astype(jnp.bfloat16)
        Cm16 = Cm.astype(jnp.bfloat16)

        for p in range(_POINTS):
            term = (jnp.dot(Rm16, Eps[p], preferred_element_type=jnp.float32)
                    * jnp.dot(Cm16, Tps[p], preferred_element_type=jnp.float32))
            if p == 0:
                A_ref[...] = term
            else:
                A_ref[...] += term

        v_h = v16[:, _HD * hh:_HD * (hh + 1)]
        out_h = jnp.dot(A_ref[...].astype(jnp.bfloat16), v_h,
                        preferred_element_type=jnp.float32)
        part = jnp.dot(out_h, WpT_ref[_HD * hh:_HD * (hh + 1), :],
                       preferred_element_type=jnp.float32)
        acc = part if acc is None else acc + part

    out_ref[0] = acc + bp_ref[...] + 2.0 * qk


@jax.jit
def kernel(query, value, Wv, bv, Wo, bo, Wa, ba, Wp, bp):
    b, c, h, w = query.shape
    nq = h * w
    qf = query.reshape(b, c, nq).transpose(0, 2, 1)
    vf = value.reshape(b, c, nq).transpose(0, 2, 1)
    permo = jnp.asarray(_PERMO)
    WoT = Wo.T[:, permo]
    boP = bo[permo].reshape(1, -1)

    bcast = lambda i: (0, 0)
    out = pl.pallas_call(
        _body,
        grid=(b,),
        in_specs=[
            pl.BlockSpec((1, _NQ, _C), lambda i: (i, 0, 0)),
            pl.BlockSpec((1, _NQ, _C), lambda i: (i, 0, 0)),
            pl.BlockSpec((_NQ, _C), bcast),
            pl.BlockSpec((_NQ, 2), bcast),
            pl.BlockSpec((_C, _C), bcast),
            pl.BlockSpec((1, _C), bcast),
            pl.BlockSpec((_C, _HEADS * _POINTS * 2), bcast),
            pl.BlockSpec((1, _HEADS * _POINTS * 2), bcast),
            pl.BlockSpec((_C, _HEADS * _POINTS), bcast),
            pl.BlockSpec((1, _HEADS * _POINTS), bcast),
            pl.BlockSpec((_C, _C), bcast),
            pl.BlockSpec((1, _C), bcast),
            pl.BlockSpec((_POINTS, 4 * _W), bcast),
            pl.BlockSpec((_HEADS * _POINTS, _HEADS * _POINTS), bcast),
            pl.BlockSpec((4 * _H, _NQ), bcast),
            pl.BlockSpec((4 * _H, _NQ), bcast),
            pl.BlockSpec((4 * _H, _NQ), bcast),
            pl.BlockSpec((4 * _H, _NQ), bcast),
            pl.BlockSpec((4 * _W, _NQ), bcast),
            pl.BlockSpec((4 * _W, _NQ), bcast),
            pl.BlockSpec((4 * _W, _NQ), bcast),
            pl.BlockSpec((4 * _W, _NQ), bcast),
        ],
        out_specs=pl.BlockSpec((1, _NQ, _C), lambda i: (i, 0, 0)),
        out_shape=jax.ShapeDtypeStruct((b, _NQ, _C), jnp.float32),
        scratch_shapes=[pltpu.VMEM((_NQ, _NQ), jnp.float32)],
        compiler_params=pltpu.CompilerParams(
            dimension_semantics=("parallel",)),
    )(qf, vf, jnp.asarray(_PE), jnp.asarray(_REF),
      Wv.T, bv.reshape(1, -1), WoT, boP,
      Wa.T, ba.reshape(1, -1), Wp.T, bp.reshape(1, -1),
      jnp.asarray(_E4B), jnp.asarray(_S4),
      *[jnp.asarray(e, dtype=jnp.bfloat16) for e in _EPS],
      *[jnp.asarray(t, dtype=jnp.bfloat16) for t in _TPS])

    return out.transpose(0, 2, 1).reshape(b, c, h, w)
